# baseline (speedup 1.0000x reference)
import jax
import jax.numpy as jnp
from jax.experimental import pallas as pl
from jax.experimental.pallas import tpu as pltpu

_GAMMA = 0.5
_SMOOTH = 1.0
_B = 64
_N = 512
_W = 512
_STEPS = 4
_SKEW = 43


def _fused_kernel(c_ref, g_ref, s_ref, ptr_ref, mtr_ref,
                  part_ref, cpart_ref, pfull_ref, acc_ref):
    j = pl.program_id(1)

    @pl.when(j == 0)
    def _():
        pbase = ptr_ref[...]
        pfull_ref[:, 0:128] = pbase
        pfull_ref[:, 128:256] = jnp.concatenate(
            [pbase[128:], pbase[:128]], axis=0)
        pfull_ref[:, 256:384] = jnp.concatenate(
            [pbase[256:], pbase[:256]], axis=0)
        pfull_ref[:, 384:512] = jnp.concatenate(
            [pbase[384:], pbase[:384]], axis=0)
        acc_ref[...] = jnp.zeros((_N, 128), jnp.float32)

    c = c_ref[...]
    g = g_ref[...]
    s = s_ref[...]
    ones_row = jnp.ones((1, c.shape[0]), jnp.float32)
    sc = jnp.dot(ones_row, c, preferred_element_type=jnp.float32)
    sg = jnp.dot(ones_row, g, preferred_element_type=jnp.float32)
    ss = jnp.dot(ones_row, s, preferred_element_type=jnp.float32)
    scg = jnp.dot(ones_row, c * g, preferred_element_type=jnp.float32)
    scs = jnp.dot(ones_row, c * s, preferred_element_type=jnp.float32)
    block = jnp.concatenate(
        [sc, sg, ss, scg, scs, jnp.zeros((3, _W), jnp.float32)], axis=0)[None]

    @pl.when(j == 0)
    def _():
        part_ref[...] = block

    @pl.when(j != 0)
    def _():
        part_ref[...] = part_ref[...] + block

    start = jnp.minimum(j * _SKEW, 128)
    stop = jnp.minimum((j + 1) * _SKEW, 128)

    def body(it, acc_val):
        w = 127 - (start + it)
        m = mtr_ref[pl.ds(w, 1), 0, :]
        t = m * pfull_ref[...]
        v = (t[:, 0:128] + t[:, 128:256]
             + t[:, 256:384] + t[:, 384:512])
        return pltpu.roll(acc_val, _N - 1, axis=0) + v

    acc_ref[...] = jax.lax.fori_loop(0, stop - start, body, acc_ref[...])

    @pl.when(j == _STEPS - 1)
    def _():
        acc = acc_ref[...]
        corr = acc[:, 0:64] + acc[:, 64:128]
        cmax = jnp.max(corr, axis=0, keepdims=True)

        pbase = ptr_ref[...]
        pq = jnp.sum(pbase * pbase, axis=0, keepdims=True)
        ap = pq[:, 0:64] + pq[:, 64:128]
        msq = mtr_ref[...][:, 0, :]
        mq = jnp.sum(msq * msq, axis=0, keepdims=True)
        mu = (mq[:, 0:128] + mq[:, 128:256]
              + mq[:, 256:384] + mq[:, 384:512])
        ar = mu[:, 0:64] + mu[:, 64:128]

        mse = (ap + ar - 2.0 * cmax) * (1.0 / (2.0 * _N))
        cpart_ref[...] = jnp.concatenate(
            [mse, jnp.zeros((1, 64), jnp.float32)], axis=1)


def _final_kernel(part_ref, cpart_ref, out_ref):
    p5 = part_ref[0] + part_ref[1]
    s_c = jnp.sum(p5[0:1, :])
    s_g = jnp.sum(p5[1:2, :])
    s_s = jnp.sum(p5[2:3, :])
    s_cg = jnp.sum(p5[3:4, :])
    s_cs = jnp.sum(p5[4:5, :])
    seg = jnp.sum(cpart_ref[0][:, 0:64]) * (1.0 / _B)
    cons = jnp.sum(cpart_ref[1][:, 0:64]) * (1.0 / _B)
    dice1 = 1.0 - (2.0 * s_cg + _SMOOTH) / (s_c + s_g + _SMOOTH)
    dice2 = 1.0 - (2.0 * s_cs + _SMOOTH) / (s_c + s_s + _SMOOTH)
    loss = (1.0 - _GAMMA) * (dice1 + seg) + _GAMMA * (dice2 + cons)
    out_ref[...] = jnp.reshape(loss, (1, 1))


def kernel(ground_truth_mask, ground_truth_contour, snake_GT_size,
           snake_classic_size, snake_mask, classic_contour, classic_mask):
    B, N, W = _B, _N, _W
    c2 = classic_mask.reshape(B * 512, W)
    g2 = ground_truth_mask.reshape(B * 512, W)
    s2 = snake_mask.reshape(B * 512, W)
    rows = (B * 512) // (2 * _STEPS)

    preds = jnp.stack([snake_GT_size, snake_classic_size])
    refs = jnp.stack([ground_truth_contour, classic_contour])
    ptr = preds.transpose(0, 2, 3, 1).reshape(2, N, 2 * B)
    mtr = (refs.transpose(0, 2, 3, 1)
           .reshape(2, 4, 128, 2, B)
           .transpose(0, 2, 1, 3, 4)
           .reshape(2, 128, 1, 4 * 2 * B))

    part, cpart = pl.pallas_call(
        _fused_kernel,
        grid=(2, _STEPS),
        in_specs=[
            pl.BlockSpec((rows, W), lambda i, j: (i * _STEPS + j, 0)),
            pl.BlockSpec((rows, W), lambda i, j: (i * _STEPS + j, 0)),
            pl.BlockSpec((rows, W), lambda i, j: (i * _STEPS + j, 0)),
            pl.BlockSpec((None, N, 2 * B), lambda i, j: (i, 0, 0)),
            pl.BlockSpec((None, 128, 1, 4 * 2 * B), lambda i, j: (i, 0, 0, 0)),
        ],
        out_specs=[
            pl.BlockSpec((1, 8, W), lambda i, j: (i, 0, 0)),
            pl.BlockSpec((None, 1, 128), lambda i, j: (i, 0, 0)),
        ],
        out_shape=[
            jax.ShapeDtypeStruct((2, 8, W), jnp.float32),
            jax.ShapeDtypeStruct((2, 1, 128), jnp.float32),
        ],
        scratch_shapes=[
            pltpu.VMEM((N, 4 * 2 * B), jnp.float32),
            pltpu.VMEM((N, 2 * B), jnp.float32),
        ],
        compiler_params=pltpu.CompilerParams(
            dimension_semantics=("parallel", "arbitrary")),
    )(c2, g2, s2, ptr, mtr)

    out = pl.pallas_call(
        _final_kernel,
        in_specs=[
            pl.BlockSpec((2, 8, W), lambda: (0, 0, 0)),
            pl.BlockSpec((2, 1, 128), lambda: (0, 0, 0)),
        ],
        out_specs=pl.BlockSpec((1, 1), lambda: (0, 0)),
        out_shape=jax.ShapeDtypeStruct((1, 1), jnp.float32),
    )(part, cpart)
    return out[0, 0]


# speedup vs baseline: 1.3796x; 1.3796x over previous
import jax
import jax.numpy as jnp
from jax.experimental import pallas as pl
from jax.experimental.pallas import tpu as pltpu

_GAMMA = 0.5
_SMOOTH = 1.0
_B = 64
_N = 512
_W = 512
_STEPS = 4
_SKEW = 44


def _fused_kernel(c_ref, g_ref, s_ref, ptr_ref, mtr_ref,
                  part_ref, cpart_ref, pfull_ref, acc_ref):
    j = pl.program_id(1)

    @pl.when(j == 0)
    def _():
        pbase = ptr_ref[...]
        pfull_ref[:, 0:128] = pbase
        pfull_ref[:, 128:256] = jnp.concatenate(
            [pbase[128:], pbase[:128]], axis=0)
        pfull_ref[:, 256:384] = jnp.concatenate(
            [pbase[256:], pbase[:256]], axis=0)
        pfull_ref[:, 384:512] = jnp.concatenate(
            [pbase[384:], pbase[:384]], axis=0)
        acc_ref[...] = jnp.zeros((_N, 128), jnp.float32)

    c = c_ref[...]
    g = g_ref[...]
    s = s_ref[...]
    ones_row = jnp.ones((1, c.shape[0]), jnp.float32)
    sc = jnp.dot(ones_row, c, preferred_element_type=jnp.float32)
    sg = jnp.dot(ones_row, g, preferred_element_type=jnp.float32)
    ss = jnp.dot(ones_row, s, preferred_element_type=jnp.float32)
    scg = jnp.dot(ones_row, c * g, preferred_element_type=jnp.float32)
    scs = jnp.dot(ones_row, c * s, preferred_element_type=jnp.float32)
    block = jnp.concatenate(
        [sc, sg, ss, scg, scs, jnp.zeros((3, _W), jnp.float32)], axis=0)[None]

    @pl.when(j == 0)
    def _():
        part_ref[...] = block

    @pl.when(j != 0)
    def _():
        part_ref[...] = part_ref[...] + block

    start = jnp.minimum(j * _SKEW, 128)
    stop = jnp.minimum((j + 1) * _SKEW, 128)

    def body(it, carry):
        w0 = 127 - (start + 2 * it)
        m0 = mtr_ref[pl.ds(w0, 1), 0, :]
        m1 = mtr_ref[pl.ds(w0 - 1, 1), 0, :]
        t0 = m0 * pfull_ref[...]
        t1 = m1 * pfull_ref[...]
        v0 = (t0[:, 0:128] + t0[:, 128:256]
              + t0[:, 256:384] + t0[:, 384:512])
        v1 = (t1[:, 0:128] + t1[:, 128:256]
              + t1[:, 256:384] + t1[:, 384:512])
        acc_ref[...] = (pltpu.roll(acc_ref[...], _N - 2, axis=0)
                        + pltpu.roll(v0, _N - 1, axis=0) + v1)
        return carry

    jax.lax.fori_loop(0, (stop - start) // 2, body, 0)

    @pl.when(j == _STEPS - 1)
    def _():
        acc = acc_ref[...]
        corr = acc[:, 0:64] + acc[:, 64:128]
        cmax = jnp.max(corr, axis=0, keepdims=True)

        pbase = ptr_ref[...]
        pq = jnp.sum(pbase * pbase, axis=0, keepdims=True)
        ap = pq[:, 0:64] + pq[:, 64:128]
        msq = mtr_ref[...][:, 0, :]
        mq = jnp.sum(msq * msq, axis=0, keepdims=True)
        mu = (mq[:, 0:128] + mq[:, 128:256]
              + mq[:, 256:384] + mq[:, 384:512])
        ar = mu[:, 0:64] + mu[:, 64:128]

        mse = (ap + ar - 2.0 * cmax) * (1.0 / (2.0 * _N))
        cpart_ref[...] = jnp.concatenate(
            [mse, jnp.zeros((1, 64), jnp.float32)], axis=1)


def _final_kernel(part_ref, cpart_ref, out_ref):
    p5 = part_ref[0] + part_ref[1]
    s_c = jnp.sum(p5[0:1, :])
    s_g = jnp.sum(p5[1:2, :])
    s_s = jnp.sum(p5[2:3, :])
    s_cg = jnp.sum(p5[3:4, :])
    s_cs = jnp.sum(p5[4:5, :])
    seg = jnp.sum(cpart_ref[0][:, 0:64]) * (1.0 / _B)
    cons = jnp.sum(cpart_ref[1][:, 0:64]) * (1.0 / _B)
    dice1 = 1.0 - (2.0 * s_cg + _SMOOTH) / (s_c + s_g + _SMOOTH)
    dice2 = 1.0 - (2.0 * s_cs + _SMOOTH) / (s_c + s_s + _SMOOTH)
    loss = (1.0 - _GAMMA) * (dice1 + seg) + _GAMMA * (dice2 + cons)
    out_ref[...] = jnp.reshape(loss, (1, 1))


def kernel(ground_truth_mask, ground_truth_contour, snake_GT_size,
           snake_classic_size, snake_mask, classic_contour, classic_mask):
    B, N, W = _B, _N, _W
    c2 = classic_mask.reshape(B * 512, W)
    g2 = ground_truth_mask.reshape(B * 512, W)
    s2 = snake_mask.reshape(B * 512, W)
    rows = (B * 512) // (2 * _STEPS)

    preds = jnp.stack([snake_GT_size, snake_classic_size])
    refs = jnp.stack([ground_truth_contour, classic_contour])
    ptr = preds.transpose(0, 2, 3, 1).reshape(2, N, 2 * B)
    mtr = (refs.transpose(0, 2, 3, 1)
           .reshape(2, 4, 128, 2, B)
           .transpose(0, 2, 1, 3, 4)
           .reshape(2, 128, 1, 4 * 2 * B))

    part, cpart = pl.pallas_call(
        _fused_kernel,
        grid=(2, _STEPS),
        in_specs=[
            pl.BlockSpec((rows, W), lambda i, j: (i * _STEPS + j, 0)),
            pl.BlockSpec((rows, W), lambda i, j: (i * _STEPS + j, 0)),
            pl.BlockSpec((rows, W), lambda i, j: (i * _STEPS + j, 0)),
            pl.BlockSpec((None, N, 2 * B), lambda i, j: (i, 0, 0)),
            pl.BlockSpec((None, 128, 1, 4 * 2 * B), lambda i, j: (i, 0, 0, 0)),
        ],
        out_specs=[
            pl.BlockSpec((1, 8, W), lambda i, j: (i, 0, 0)),
            pl.BlockSpec((None, 1, 128), lambda i, j: (i, 0, 0)),
        ],
        out_shape=[
            jax.ShapeDtypeStruct((2, 8, W), jnp.float32),
            jax.ShapeDtypeStruct((2, 1, 128), jnp.float32),
        ],
        scratch_shapes=[
            pltpu.VMEM((N, 4 * 2 * B), jnp.float32),
            pltpu.VMEM((N, 2 * B), jnp.float32),
        ],
        compiler_params=pltpu.CompilerParams(
            dimension_semantics=("parallel", "arbitrary")),
    )(c2, g2, s2, ptr, mtr)

    out = pl.pallas_call(
        _final_kernel,
        in_specs=[
            pl.BlockSpec((2, 8, W), lambda: (0, 0, 0)),
            pl.BlockSpec((2, 1, 128), lambda: (0, 0, 0)),
        ],
        out_specs=pl.BlockSpec((1, 1), lambda: (0, 0)),
        out_shape=jax.ShapeDtypeStruct((1, 1), jnp.float32),
    )(part, cpart)
    return out[0, 0]
